# baseline (speedup 1.0000x reference)
import jax
import jax.numpy as jnp
from jax import lax
from jax.experimental import pallas as pl
from jax.experimental.pallas import tpu as pltpu
from jax.experimental.pallas import tpu_sc as plsc

_B = 4096
_F = 50 * 128
_NC, _NS = 2, 16
_NW = _NC * _NS
_RPW = _B // _NW


def _sc_roll_body(x_ref, o_ref):
    wid = lax.axis_index("s") * _NC + lax.axis_index("c")
    base = wid * _RPW
    s = base * _F

    @pl.when(wid != _NW - 1)
    def _():
        pltpu.sync_copy(x_ref.at[pl.ds(s + _F, _RPW * _F)],
                        o_ref.at[pl.ds(s, _RPW * _F)])

    @pl.when(wid == _NW - 1)
    def _():
        pltpu.sync_copy(x_ref.at[pl.ds(s + _F, (_RPW - 1) * _F)],
                        o_ref.at[pl.ds(s, (_RPW - 1) * _F)])
        pltpu.sync_copy(x_ref.at[pl.ds(0, _F)],
                        o_ref.at[pl.ds((_B - 1) * _F, _F)])


def kernel(x):
    x1 = x.reshape(_B * _F)
    out = pl.kernel(
        _sc_roll_body,
        out_type=jax.ShapeDtypeStruct((_B * _F,), jnp.float32),
        mesh=plsc.VectorSubcoreMesh(core_axis_name="c", subcore_axis_name="s"),
    )(x1)
    return out.reshape(x.shape)


# speedup vs baseline: 7.6141x; 7.6141x over previous
import jax
import jax.numpy as jnp
from jax import lax
from jax.experimental import pallas as pl
from jax.experimental.pallas import tpu as pltpu
from jax.experimental.pallas import tpu_sc as plsc

_B = 4096
_F = 50 * 128
_N = _B * _F
_NC, _NS = 2, 16
_NW = _NC * _NS
_RPW = _B // _NW
_CH = 8
_NCHUNK = _RPW // _CH
_NBUF = 2


def _sc_roll_body(x_ref, o_ref, buf0, buf1, rs0, rs1, ws0, ws1):
    bufs, rsems, wsems = [buf0, buf1], [rs0, rs1], [ws0, ws1]
    wid = lax.axis_index("s") * _NC + lax.axis_index("c")
    base = wid * _RPW

    def issue_read(c):
        b = c % _NBUF
        s = (base + c * _CH) * _F
        if c < _NCHUNK - 1:
            return [pltpu.async_copy(x_ref.at[pl.ds(s + _F, _CH * _F)],
                                     bufs[b], rsems[b])]
        src2 = lax.rem(s + _CH * _F, _N)
        return [
            pltpu.async_copy(x_ref.at[pl.ds(s + _F, (_CH - 1) * _F)],
                             bufs[b].at[pl.ds(0, (_CH - 1) * _F)], rsems[b]),
            pltpu.async_copy(x_ref.at[pl.ds(src2, _F)],
                             bufs[b].at[pl.ds((_CH - 1) * _F, _F)], rsems[b]),
        ]

    def issue_write(c):
        b = c % _NBUF
        s = (base + c * _CH) * _F
        return [pltpu.async_copy(bufs[b], o_ref.at[pl.ds(s, _CH * _F)],
                                 wsems[b])]

    reads, writes = {}, {}
    reads[0] = issue_read(0)
    for c in range(_NCHUNK):
        nxt = c + 1
        if nxt < _NCHUNK:
            if nxt >= _NBUF:
                for h in writes[nxt - _NBUF]:
                    h.wait()
            reads[nxt] = issue_read(nxt)
        for h in reads[c]:
            h.wait()
        writes[c] = issue_write(c)
    for c in range(_NCHUNK - _NBUF, _NCHUNK):
        for h in writes[c]:
            h.wait()


def kernel(x):
    x1 = x.reshape(_N)
    out = pl.kernel(
        _sc_roll_body,
        out_type=jax.ShapeDtypeStruct((_N,), jnp.float32),
        mesh=plsc.VectorSubcoreMesh(core_axis_name="c", subcore_axis_name="s"),
        scratch_types=[pltpu.VMEM((_CH * _F,), jnp.float32)] * _NBUF
                      + [pltpu.SemaphoreType.DMA] * (2 * _NBUF),
    )(x1)
    return out.reshape(x.shape)


# speedup vs baseline: 15.7212x; 2.0647x over previous
import jax
import jax.numpy as jnp
from jax import lax
from jax.experimental import pallas as pl
from jax.experimental.pallas import tpu as pltpu
from jax.experimental.pallas import tpu_sc as plsc

_B = 4096
_S, _L = 50, 128
_NC, _NS = 2, 16
_NW = _NC * _NS
_RPW = _B // _NW
_CH = 8
_NCHUNK = _RPW // _CH
_NBUF = 2


def _sc_roll_body(x_ref, o_ref, buf0, buf1, rs0, rs1, ws0, ws1):
    bufs, rsems, wsems = [buf0, buf1], [rs0, rs1], [ws0, ws1]
    wid = lax.axis_index("s") * _NC + lax.axis_index("c")
    base = wid * _RPW

    def issue_read(c):
        b = c % _NBUF
        s = base + c * _CH
        if c < _NCHUNK - 1:
            return [pltpu.async_copy(x_ref.at[pl.ds(s + 1, _CH)],
                                     bufs[b], rsems[b])]
        src2 = lax.rem(s + _CH, _B)
        return [
            pltpu.async_copy(x_ref.at[pl.ds(s + 1, _CH - 1)],
                             bufs[b].at[pl.ds(0, _CH - 1)], rsems[b]),
            pltpu.async_copy(x_ref.at[pl.ds(src2, 1)],
                             bufs[b].at[pl.ds(_CH - 1, 1)], rsems[b]),
        ]

    def issue_write(c):
        b = c % _NBUF
        s = base + c * _CH
        return [pltpu.async_copy(bufs[b], o_ref.at[pl.ds(s, _CH)], wsems[b])]

    reads, writes = {}, {}
    reads[0] = issue_read(0)
    for c in range(_NCHUNK):
        nxt = c + 1
        if nxt < _NCHUNK:
            if nxt >= _NBUF:
                for h in writes[nxt - _NBUF]:
                    h.wait()
            reads[nxt] = issue_read(nxt)
        for h in reads[c]:
            h.wait()
        writes[c] = issue_write(c)
    for c in range(_NCHUNK - _NBUF, _NCHUNK):
        for h in writes[c]:
            h.wait()


def kernel(x):
    return pl.kernel(
        _sc_roll_body,
        out_type=jax.ShapeDtypeStruct((_B, _S, _L), jnp.float32),
        mesh=plsc.VectorSubcoreMesh(core_axis_name="c", subcore_axis_name="s"),
        scratch_types=[pltpu.VMEM((_CH, _S, _L), jnp.float32)] * _NBUF
                      + [pltpu.SemaphoreType.DMA] * (2 * _NBUF),
    )(x)
